# baseline (speedup 1.0000x reference)
import functools

import jax
import jax.numpy as jnp
from jax import lax
from jax.experimental import pallas as pl
from jax.experimental.pallas import tpu as pltpu
from jax.experimental.pallas import tpu_sc as plsc

NSEG = 64
FEAT = 128
LANE = 16
NVEC = FEAT // LANE
NCORE = 2
NSUB = 16
NW = NCORE * NSUB
RCHUNK = 128
NBUF = 4

SIZES = (100000, 50000, 25000)


def _cfg(n):
    c = -(-n // NW)
    c = -(-c // 8) * 8
    nch = -(-c // RCHUNK)
    return c, nch


_CFGS = tuple(_cfg(n) for n in SIZES)


def _sc_body(x0, x1, x2, s0, s1, s2,
             ps0, pm0, pc0, ps1, pm1, pc1, ps2, pm2, pc2,
             buf0, buf1, buf2, buf3, sg0, sg1, sg2, lsum, lmax, lcnt,
             accv, curs, sem0, sem1, sem2, sem3):
    wid = lax.axis_index("c") * NSUB + lax.axis_index("s")
    bufs = (buf0, buf1, buf2, buf3)
    sems = (sem0, sem1, sem2, sem3)
    zero = jnp.zeros((LANE,), jnp.float32)
    ninf = jnp.full((LANE,), -jnp.inf, jnp.float32)
    one = jnp.ones((LANE,), jnp.float32)

    layers = (
        (x0, s0, sg0, ps0, pm0, pc0, SIZES[0], _CFGS[0]),
        (x1, s1, sg1, ps1, pm1, pc1, SIZES[1], _CFGS[1]),
        (x2, s2, sg2, ps2, pm2, pc2, SIZES[2], _CFGS[2]),
    )

    NB = RCHUNK // LANE

    for (x, seg, segv, psum, pmax, pcnt, n, (c, nch)) in layers:
        def _init(i, _):
            lsum[pl.ds(i * LANE, LANE)] = zero
            lmax[pl.ds(i * LANE, LANE)] = ninf
            return 0
        lax.fori_loop(0, NSEG * NVEC, _init, 0)

        def _initc(i, _):
            lcnt[pl.ds(i * LANE, LANE)] = zero
            return 0
        lax.fori_loop(0, NSEG, _initc, 0)

        start = wid * c
        cnt = jnp.minimum(c, n - start)
        sstart = jnp.minimum(start, n - c)
        pltpu.sync_copy(seg.at[pl.ds(sstart, c)], segv.at[pl.ds(0, c)])

        def _ceff(j):
            return jnp.minimum(start + j * RCHUNK, n - RCHUNK)

        def _flush_reset(new_seg):
            cur = curs[0]
            for k in range(NVEC):
                lsum[pl.ds(cur * FEAT + LANE * k, LANE)] = \
                    accv[pl.ds(LANE * k, LANE)]
                lmax[pl.ds(cur * FEAT + LANE * k, LANE)] = \
                    accv[pl.ds(FEAT + LANE * k, LANE)]
                accv[pl.ds(LANE * k, LANE)] = zero
                accv[pl.ds(FEAT + LANE * k, LANE)] = ninf
            lcnt[pl.ds(cur * LANE, LANE)] = accv[pl.ds(2 * FEAT, LANE)]
            accv[pl.ds(2 * FEAT, LANE)] = zero
            curs[0] = new_seg

        for k in range(NVEC):
            accv[pl.ds(LANE * k, LANE)] = zero
            accv[pl.ds(FEAT + LANE * k, LANE)] = ninf
        accv[pl.ds(2 * FEAT, LANE)] = zero
        curs[0] = segv[pl.ds(start - sstart, LANE)][0]

        for b in range(min(NBUF, nch)):
            pltpu.async_copy(
                x.at[pl.ds(_ceff(b) * FEAT, RCHUNK * FEAT)],
                bufs[b], sems[b])

        ngrp = -(-nch // NBUF)

        def _grp(jj, _, x=x, segv=segv, c=c, n=n, start=start,
                 cnt=cnt, sstart=sstart, nch=nch):
            for b in range(NBUF):
                j = NBUF * jj + b
                ce = _ceff(j)

                @pl.when(j < nch)
                def _():
                    pltpu.make_async_copy(
                        x.at[pl.ds(ce * FEAT, RCHUNK * FEAT)],
                        bufs[b], sems[b]).wait()

                buf = bufs[b]
                boff = start + j * RCHUNK - ce
                sidx0 = start + j * RCHUNK - sstart
                valid = jnp.clip(cnt - j * RCHUNK, 0, RCHUNK)

                def _blk(blk, _, buf=buf, boff=boff, sidx0=sidx0,
                         valid=valid):
                    rem = valid - blk * LANE
                    sldx = jnp.minimum(sidx0 + blk * LANE, c)
                    svec = segv[pl.ds(sldx, LANE)]
                    s = svec[0]
                    fast = jnp.logical_and(svec[LANE - 1] == s, rem >= LANE)

                    @pl.when(fast)
                    def _():
                        @pl.when(s != curs[0])
                        def _():
                            _flush_reset(s)

                        accs = [zero] * NVEC
                        accm = [accv[pl.ds(FEAT + LANE * k, LANE)]
                                for k in range(NVEC)]
                        base = (boff + blk * LANE) * FEAT
                        for i in range(LANE):
                            for k in range(NVEC):
                                r = buf[pl.ds(base + i * FEAT + k * LANE,
                                              LANE)]
                                accs[k] = accs[k] + r
                                accm[k] = jnp.maximum(accm[k], r)
                        for k in range(NVEC):
                            plsc.addupdate(
                                accv.at[pl.ds(LANE * k, LANE)], accs[k])
                            accv[pl.ds(FEAT + LANE * k, LANE)] = accm[k]
                        plsc.addupdate(
                            accv.at[pl.ds(2 * FEAT, LANE)],
                            jnp.full((LANE,), float(LANE), jnp.float32))

                    @pl.when(jnp.logical_not(fast))
                    def _():
                        def _row(r, _):
                            sr = segv[pl.ds(sidx0 + blk * LANE + r,
                                            LANE)][0]

                            @pl.when(sr != curs[0])
                            def _():
                                _flush_reset(sr)

                            base = (boff + blk * LANE + r) * FEAT
                            for k in range(NVEC):
                                rv = buf[pl.ds(base + LANE * k, LANE)]
                                plsc.addupdate(
                                    accv.at[pl.ds(LANE * k, LANE)], rv)
                                accv[pl.ds(FEAT + LANE * k, LANE)] = \
                                    jnp.maximum(
                                        accv[pl.ds(FEAT + LANE * k, LANE)],
                                        rv)
                            plsc.addupdate(
                                accv.at[pl.ds(2 * FEAT, LANE)], one)
                            return 0

                        lax.fori_loop(0, jnp.clip(rem, 0, LANE), _row, 0)

                    return 0

                @pl.when(valid > 0)
                def _():
                    lax.fori_loop(0, NB, _blk, 0)

                cr = _ceff(j + NBUF)

                @pl.when(j + NBUF < nch)
                def _():
                    pltpu.async_copy(
                        x.at[pl.ds(cr * FEAT, RCHUNK * FEAT)],
                        bufs[b], sems[b])
            return 0

        lax.fori_loop(0, ngrp, _grp, 0)

        cur = curs[0]
        for k in range(NVEC):
            lsum[pl.ds(cur * FEAT + LANE * k, LANE)] = \
                accv[pl.ds(LANE * k, LANE)]
            lmax[pl.ds(cur * FEAT + LANE * k, LANE)] = \
                accv[pl.ds(FEAT + LANE * k, LANE)]
        lcnt[pl.ds(cur * LANE, LANE)] = accv[pl.ds(2 * FEAT, LANE)]

        pltpu.sync_copy(lsum, psum.at[wid])
        pltpu.sync_copy(lmax, pmax.at[wid])
        pltpu.sync_copy(lcnt, pcnt.at[wid])


@functools.cache
def _make_sc_call():
    mesh = plsc.VectorSubcoreMesh(core_axis_name="c", subcore_axis_name="s")
    f32 = jnp.float32
    outs = []
    for _n in SIZES:
        outs += [jax.ShapeDtypeStruct((NW, NSEG * FEAT), f32),
                 jax.ShapeDtypeStruct((NW, NSEG * FEAT), f32),
                 jax.ShapeDtypeStruct((NW, NSEG * LANE), f32)]
    scratch = [
        pltpu.VMEM((RCHUNK * FEAT,), f32),
        pltpu.VMEM((RCHUNK * FEAT,), f32),
        pltpu.VMEM((RCHUNK * FEAT,), f32),
        pltpu.VMEM((RCHUNK * FEAT,), f32),
        pltpu.VMEM((_CFGS[0][0] + LANE,), jnp.int32),
        pltpu.VMEM((_CFGS[1][0] + LANE,), jnp.int32),
        pltpu.VMEM((_CFGS[2][0] + LANE,), jnp.int32),
        pltpu.VMEM((NSEG * FEAT,), f32),
        pltpu.VMEM((NSEG * FEAT,), f32),
        pltpu.VMEM((NSEG * LANE,), f32),
        pltpu.VMEM((2 * FEAT + LANE,), f32),
        pltpu.SMEM((1,), jnp.int32),
        pltpu.SemaphoreType.DMA,
        pltpu.SemaphoreType.DMA,
        pltpu.SemaphoreType.DMA,
        pltpu.SemaphoreType.DMA,
    ]
    return pl.kernel(_sc_body, mesh=mesh, out_type=outs, scratch_types=scratch)


def _merge_body(ps0, pm0, pc0, ps1, pm1, pc1, ps2, pm2, pc2, out_ref):
    parts = ((ps0, pm0, pc0), (ps1, pm1, pc1), (ps2, pm2, pc2))
    for i, (ps, pm, pc) in enumerate(parts):
        s = jnp.sum(ps[...], axis=0)
        m = jnp.max(pm[...], axis=0)
        cvec = jnp.sum(pc[...], axis=0)
        cnt = cvec[:, 0:1]
        mean = s / jnp.maximum(cnt, 1.0)
        m = jnp.where(cnt > 0.0, m, 0.0)
        out_ref[:, i * FEAT:(i + 1) * FEAT] = m
        out_ref[:, 3 * FEAT + i * FEAT:3 * FEAT + (i + 1) * FEAT] = s
        out_ref[:, 6 * FEAT + i * FEAT:6 * FEAT + (i + 1) * FEAT] = mean


_merge = pl.pallas_call(
    _merge_body,
    out_shape=jax.ShapeDtypeStruct((NSEG, 9 * FEAT), jnp.float32),
)


def kernel(hs_0, hs_1, hs_2, gi_0, gi_1, gi_2):
    parts = _make_sc_call()(
        hs_0.reshape(-1), hs_1.reshape(-1), hs_2.reshape(-1),
        gi_2, gi_1, gi_0)
    shaped = []
    for i, p in enumerate(parts):
        if i % 3 == 2:
            shaped.append(p.reshape(NW, NSEG, LANE))
        else:
            shaped.append(p.reshape(NW, NSEG, FEAT))
    return _merge(*shaped)


# speedup vs baseline: 1.1453x; 1.1453x over previous
import functools

import jax
import jax.numpy as jnp
from jax import lax
from jax.experimental import pallas as pl
from jax.experimental.pallas import tpu as pltpu
from jax.experimental.pallas import tpu_sc as plsc

NSEG = 64
FEAT = 128
LANE = 16
NVEC = FEAT // LANE
NCORE = 2
NSUB = 16
NW = NCORE * NSUB
RCHUNK = 256
NBUF = 2

SIZES = (100000, 50000, 25000)


def _cfg(n):
    c = -(-n // NW)
    c = -(-c // 8) * 8
    nch = -(-c // RCHUNK)
    return c, nch


_CFGS = tuple(_cfg(n) for n in SIZES)


def _sc_body(x0, x1, x2, s0, s1, s2,
             ps0, pm0, pc0, ps1, pm1, pc1, ps2, pm2, pc2,
             buf0, buf1, sg0, sg1, sg2, lsum, lmax, lcnt,
             accv, curs, sem0, sem1):
    wid = lax.axis_index("c") * NSUB + lax.axis_index("s")
    bufs = (buf0, buf1)
    sems = (sem0, sem1)
    zero = jnp.zeros((LANE,), jnp.float32)
    ninf = jnp.full((LANE,), -jnp.inf, jnp.float32)
    one = jnp.ones((LANE,), jnp.float32)

    layers = (
        (x0, s0, sg0, ps0, pm0, pc0, SIZES[0], _CFGS[0]),
        (x1, s1, sg1, ps1, pm1, pc1, SIZES[1], _CFGS[1]),
        (x2, s2, sg2, ps2, pm2, pc2, SIZES[2], _CFGS[2]),
    )

    NB = RCHUNK // LANE

    for (x, seg, segv, psum, pmax, pcnt, n, (c, nch)) in layers:
        def _init(i, _):
            lsum[pl.ds(i * LANE, LANE)] = zero
            lmax[pl.ds(i * LANE, LANE)] = ninf
            return 0
        lax.fori_loop(0, NSEG * NVEC, _init, 0)

        def _initc(i, _):
            lcnt[pl.ds(i * LANE, LANE)] = zero
            return 0
        lax.fori_loop(0, NSEG, _initc, 0)

        start = wid * c
        cnt = jnp.minimum(c, n - start)
        sstart = jnp.minimum(start, n - c)
        pltpu.sync_copy(seg.at[pl.ds(sstart, c)], segv.at[pl.ds(0, c)])

        def _ceff(j):
            return jnp.minimum(start + j * RCHUNK, n - RCHUNK)

        def _flush_reset(new_seg):
            cur = curs[0]
            for k in range(NVEC):
                lsum[pl.ds(cur * FEAT + LANE * k, LANE)] = \
                    accv[pl.ds(LANE * k, LANE)]
                lmax[pl.ds(cur * FEAT + LANE * k, LANE)] = \
                    accv[pl.ds(FEAT + LANE * k, LANE)]
                accv[pl.ds(LANE * k, LANE)] = zero
                accv[pl.ds(FEAT + LANE * k, LANE)] = ninf
            lcnt[pl.ds(cur * LANE, LANE)] = accv[pl.ds(2 * FEAT, LANE)]
            accv[pl.ds(2 * FEAT, LANE)] = zero
            curs[0] = new_seg

        for k in range(NVEC):
            accv[pl.ds(LANE * k, LANE)] = zero
            accv[pl.ds(FEAT + LANE * k, LANE)] = ninf
        accv[pl.ds(2 * FEAT, LANE)] = zero
        curs[0] = segv[pl.ds(start - sstart, LANE)][0]

        for b in range(min(NBUF, nch)):
            pltpu.async_copy(
                x.at[pl.ds(_ceff(b) * FEAT, RCHUNK * FEAT)],
                bufs[b], sems[b])

        ngrp = -(-nch // NBUF)

        def _grp(jj, _, x=x, segv=segv, c=c, n=n, start=start,
                 cnt=cnt, sstart=sstart, nch=nch):
            for b in range(NBUF):
                j = NBUF * jj + b
                ce = _ceff(j)

                @pl.when(j < nch)
                def _():
                    pltpu.make_async_copy(
                        x.at[pl.ds(ce * FEAT, RCHUNK * FEAT)],
                        bufs[b], sems[b]).wait()

                buf = bufs[b]
                boff = start + j * RCHUNK - ce
                sidx0 = start + j * RCHUNK - sstart
                valid = jnp.clip(cnt - j * RCHUNK, 0, RCHUNK)

                sa = segv[pl.ds(jnp.minimum(sidx0, c), LANE)][0]
                sb = segv[pl.ds(jnp.clip(sidx0 + valid - 1, 0, c),
                                LANE)][0]
                uni = jnp.logical_and(valid == RCHUNK, sa == sb)

                @pl.when(uni)
                def _(buf=buf, boff=boff, sa=sa):
                    @pl.when(sa != curs[0])
                    def _():
                        _flush_reset(sa)

                    accm0 = tuple(accv[pl.ds(FEAT + LANE * k, LANE)]
                                  for k in range(NVEC))
                    accs0 = tuple([zero] * NVEC)

                    def _ublk(blk, carry, buf=buf, boff=boff):
                        accs, accm = carry
                        accs, accm = list(accs), list(accm)
                        base = (boff + blk * LANE) * FEAT
                        for i in range(LANE):
                            for k in range(NVEC):
                                r = buf[pl.ds(base + i * FEAT + k * LANE,
                                              LANE)]
                                accs[k] = accs[k] + r
                                accm[k] = jnp.maximum(accm[k], r)
                        return (tuple(accs), tuple(accm))

                    accs, accm = lax.fori_loop(
                        0, NB, _ublk, (accs0, accm0))
                    for k in range(NVEC):
                        plsc.addupdate(
                            accv.at[pl.ds(LANE * k, LANE)], accs[k])
                        accv[pl.ds(FEAT + LANE * k, LANE)] = accm[k]
                    plsc.addupdate(
                        accv.at[pl.ds(2 * FEAT, LANE)],
                        jnp.full((LANE,), float(RCHUNK), jnp.float32))

                def _blk(blk, _, buf=buf, boff=boff, sidx0=sidx0,
                         valid=valid):
                    rem = valid - blk * LANE
                    sldx = jnp.minimum(sidx0 + blk * LANE, c)
                    svec = segv[pl.ds(sldx, LANE)]
                    s = svec[0]
                    fast = jnp.logical_and(svec[LANE - 1] == s, rem >= LANE)

                    @pl.when(fast)
                    def _():
                        @pl.when(s != curs[0])
                        def _():
                            _flush_reset(s)

                        accs = [zero] * NVEC
                        accm = [accv[pl.ds(FEAT + LANE * k, LANE)]
                                for k in range(NVEC)]
                        base = (boff + blk * LANE) * FEAT
                        for i in range(LANE):
                            for k in range(NVEC):
                                r = buf[pl.ds(base + i * FEAT + k * LANE,
                                              LANE)]
                                accs[k] = accs[k] + r
                                accm[k] = jnp.maximum(accm[k], r)
                        for k in range(NVEC):
                            plsc.addupdate(
                                accv.at[pl.ds(LANE * k, LANE)], accs[k])
                            accv[pl.ds(FEAT + LANE * k, LANE)] = accm[k]
                        plsc.addupdate(
                            accv.at[pl.ds(2 * FEAT, LANE)],
                            jnp.full((LANE,), float(LANE), jnp.float32))

                    @pl.when(jnp.logical_not(fast))
                    def _():
                        def _row(r, _):
                            sr = segv[pl.ds(sidx0 + blk * LANE + r,
                                            LANE)][0]

                            @pl.when(sr != curs[0])
                            def _():
                                _flush_reset(sr)

                            base = (boff + blk * LANE + r) * FEAT
                            for k in range(NVEC):
                                rv = buf[pl.ds(base + LANE * k, LANE)]
                                plsc.addupdate(
                                    accv.at[pl.ds(LANE * k, LANE)], rv)
                                accv[pl.ds(FEAT + LANE * k, LANE)] = \
                                    jnp.maximum(
                                        accv[pl.ds(FEAT + LANE * k, LANE)],
                                        rv)
                            plsc.addupdate(
                                accv.at[pl.ds(2 * FEAT, LANE)], one)
                            return 0

                        lax.fori_loop(0, jnp.clip(rem, 0, LANE), _row, 0)

                    return 0

                @pl.when(jnp.logical_and(jnp.logical_not(uni), valid > 0))
                def _():
                    lax.fori_loop(0, NB, _blk, 0)

                cr = _ceff(j + NBUF)

                @pl.when(j + NBUF < nch)
                def _():
                    pltpu.async_copy(
                        x.at[pl.ds(cr * FEAT, RCHUNK * FEAT)],
                        bufs[b], sems[b])
            return 0

        lax.fori_loop(0, ngrp, _grp, 0)

        cur = curs[0]
        for k in range(NVEC):
            lsum[pl.ds(cur * FEAT + LANE * k, LANE)] = \
                accv[pl.ds(LANE * k, LANE)]
            lmax[pl.ds(cur * FEAT + LANE * k, LANE)] = \
                accv[pl.ds(FEAT + LANE * k, LANE)]
        lcnt[pl.ds(cur * LANE, LANE)] = accv[pl.ds(2 * FEAT, LANE)]

        pltpu.sync_copy(lsum, psum.at[wid])
        pltpu.sync_copy(lmax, pmax.at[wid])
        pltpu.sync_copy(lcnt, pcnt.at[wid])


@functools.cache
def _make_sc_call():
    mesh = plsc.VectorSubcoreMesh(core_axis_name="c", subcore_axis_name="s")
    f32 = jnp.float32
    outs = []
    for _n in SIZES:
        outs += [jax.ShapeDtypeStruct((NW, NSEG * FEAT), f32),
                 jax.ShapeDtypeStruct((NW, NSEG * FEAT), f32),
                 jax.ShapeDtypeStruct((NW, NSEG * LANE), f32)]
    scratch = [
        pltpu.VMEM((RCHUNK * FEAT,), f32),
        pltpu.VMEM((RCHUNK * FEAT,), f32),
        pltpu.VMEM((_CFGS[0][0] + LANE,), jnp.int32),
        pltpu.VMEM((_CFGS[1][0] + LANE,), jnp.int32),
        pltpu.VMEM((_CFGS[2][0] + LANE,), jnp.int32),
        pltpu.VMEM((NSEG * FEAT,), f32),
        pltpu.VMEM((NSEG * FEAT,), f32),
        pltpu.VMEM((NSEG * LANE,), f32),
        pltpu.VMEM((2 * FEAT + LANE,), f32),
        pltpu.SMEM((1,), jnp.int32),
        pltpu.SemaphoreType.DMA,
        pltpu.SemaphoreType.DMA,
    ]
    return pl.kernel(_sc_body, mesh=mesh, out_type=outs, scratch_types=scratch)


def _merge_body(ps0, pm0, pc0, ps1, pm1, pc1, ps2, pm2, pc2, out_ref):
    parts = ((ps0, pm0, pc0), (ps1, pm1, pc1), (ps2, pm2, pc2))
    for i, (ps, pm, pc) in enumerate(parts):
        s = jnp.sum(ps[...], axis=0)
        m = jnp.max(pm[...], axis=0)
        cvec = jnp.sum(pc[...], axis=0)
        cnt = cvec[:, 0:1]
        mean = s / jnp.maximum(cnt, 1.0)
        m = jnp.where(cnt > 0.0, m, 0.0)
        out_ref[:, i * FEAT:(i + 1) * FEAT] = m
        out_ref[:, 3 * FEAT + i * FEAT:3 * FEAT + (i + 1) * FEAT] = s
        out_ref[:, 6 * FEAT + i * FEAT:6 * FEAT + (i + 1) * FEAT] = mean


_merge = pl.pallas_call(
    _merge_body,
    out_shape=jax.ShapeDtypeStruct((NSEG, 9 * FEAT), jnp.float32),
)


def kernel(hs_0, hs_1, hs_2, gi_0, gi_1, gi_2):
    parts = _make_sc_call()(
        hs_0.reshape(-1), hs_1.reshape(-1), hs_2.reshape(-1),
        gi_2, gi_1, gi_0)
    shaped = []
    for i, p in enumerate(parts):
        if i % 3 == 2:
            shaped.append(p.reshape(NW, NSEG, LANE))
        else:
            shaped.append(p.reshape(NW, NSEG, FEAT))
    return _merge(*shaped)


# speedup vs baseline: 1.1678x; 1.0196x over previous
import functools

import jax
import jax.numpy as jnp
from jax import lax
from jax.experimental import pallas as pl
from jax.experimental.pallas import tpu as pltpu
from jax.experimental.pallas import tpu_sc as plsc

NSEG = 64
FEAT = 128
LANE = 16
NVEC = FEAT // LANE
NCORE = 2
NSUB = 16
NW = NCORE * NSUB
RCHUNK = 256
NBUF = 2

SIZES = (100000, 50000, 25000)


def _cfg(n):
    c = -(-n // NW)
    c = -(-c // 8) * 8
    nch = -(-c // RCHUNK)
    return c, nch


_CFGS = tuple(_cfg(n) for n in SIZES)


def _sc_body(x0, x1, x2, s0, s1, s2,
             ps0, pm0, pc0, ps1, pm1, pc1, ps2, pm2, pc2,
             buf0, buf1, sg0, sg1, sg2,
             lsum0, lmax0, lcnt0, lsum1, lmax1, lcnt1, lsum2, lmax2, lcnt2,
             accv, curs, sem0, sem1, psem):
    wid = lax.axis_index("c") * NSUB + lax.axis_index("s")
    bufs = (buf0, buf1)
    sems = (sem0, sem1)
    zero = jnp.zeros((LANE,), jnp.float32)
    ninf = jnp.full((LANE,), -jnp.inf, jnp.float32)
    one = jnp.ones((LANE,), jnp.float32)

    layers = (
        (x0, s0, sg0, ps0, pm0, pc0, lsum0, lmax0, lcnt0,
         SIZES[0], _CFGS[0]),
        (x1, s1, sg1, ps1, pm1, pc1, lsum1, lmax1, lcnt1,
         SIZES[1], _CFGS[1]),
        (x2, s2, sg2, ps2, pm2, pc2, lsum2, lmax2, lcnt2,
         SIZES[2], _CFGS[2]),
    )

    NB = RCHUNK // LANE

    for (_x, seg, segv, _ps, _pm, _pc, _l1, _l2, _l3, n, (c, nch)) in layers:
        start = wid * c
        sstart = jnp.minimum(start, n - c)
        pltpu.async_copy(seg.at[pl.ds(sstart, c)], segv.at[pl.ds(0, c)],
                         psem)
    for (_x, seg, segv, _ps, _pm, _pc, _l1, _l2, _l3, n, (c, nch)) in layers:
        start = wid * c
        sstart = jnp.minimum(start, n - c)
        pltpu.make_async_copy(seg.at[pl.ds(sstart, c)],
                              segv.at[pl.ds(0, c)], psem).wait()

    pending = []
    for (x, seg, segv, psum, pmax, pcnt, lsum, lmax, lcnt,
         n, (c, nch)) in layers:
        def _init(i, _):
            lsum[pl.ds(i * LANE, LANE)] = zero
            lmax[pl.ds(i * LANE, LANE)] = ninf
            return 0
        lax.fori_loop(0, NSEG * NVEC, _init, 0)

        def _initc(i, _):
            lcnt[pl.ds(i * LANE, LANE)] = zero
            return 0
        lax.fori_loop(0, NSEG, _initc, 0)

        start = wid * c
        cnt = jnp.minimum(c, n - start)
        sstart = jnp.minimum(start, n - c)

        def _ceff(j):
            return jnp.minimum(start + j * RCHUNK, n - RCHUNK)

        def _flush_reset(new_seg):
            cur = curs[0]
            for k in range(NVEC):
                lsum[pl.ds(cur * FEAT + LANE * k, LANE)] = \
                    accv[pl.ds(LANE * k, LANE)]
                lmax[pl.ds(cur * FEAT + LANE * k, LANE)] = \
                    accv[pl.ds(FEAT + LANE * k, LANE)]
                accv[pl.ds(LANE * k, LANE)] = zero
                accv[pl.ds(FEAT + LANE * k, LANE)] = ninf
            lcnt[pl.ds(cur * LANE, LANE)] = accv[pl.ds(2 * FEAT, LANE)]
            accv[pl.ds(2 * FEAT, LANE)] = zero
            curs[0] = new_seg

        for k in range(NVEC):
            accv[pl.ds(LANE * k, LANE)] = zero
            accv[pl.ds(FEAT + LANE * k, LANE)] = ninf
        accv[pl.ds(2 * FEAT, LANE)] = zero
        curs[0] = segv[pl.ds(start - sstart, LANE)][0]

        for b in range(min(NBUF, nch)):
            pltpu.async_copy(
                x.at[pl.ds(_ceff(b) * FEAT, RCHUNK * FEAT)],
                bufs[b], sems[b])

        ngrp = -(-nch // NBUF)

        def _grp(jj, _, x=x, segv=segv, c=c, n=n, start=start,
                 cnt=cnt, sstart=sstart, nch=nch):
            for b in range(NBUF):
                j = NBUF * jj + b
                ce = _ceff(j)

                @pl.when(j < nch)
                def _():
                    pltpu.make_async_copy(
                        x.at[pl.ds(ce * FEAT, RCHUNK * FEAT)],
                        bufs[b], sems[b]).wait()

                buf = bufs[b]
                boff = start + j * RCHUNK - ce
                sidx0 = start + j * RCHUNK - sstart
                valid = jnp.clip(cnt - j * RCHUNK, 0, RCHUNK)

                sa = segv[pl.ds(jnp.minimum(sidx0, c), LANE)][0]
                sb = segv[pl.ds(jnp.clip(sidx0 + valid - 1, 0, c),
                                LANE)][0]
                uni = jnp.logical_and(valid == RCHUNK, sa == sb)

                @pl.when(uni)
                def _(buf=buf, boff=boff, sa=sa):
                    @pl.when(sa != curs[0])
                    def _():
                        _flush_reset(sa)

                    accm0 = tuple(accv[pl.ds(FEAT + LANE * k, LANE)]
                                  for k in range(NVEC))
                    accs0 = tuple([zero] * NVEC)

                    def _ublk(blk, carry, buf=buf, boff=boff):
                        accs, accm = carry
                        accs, accm = list(accs), list(accm)
                        base = (boff + blk * LANE) * FEAT
                        for i in range(LANE):
                            for k in range(NVEC):
                                r = buf[pl.ds(base + i * FEAT + k * LANE,
                                              LANE)]
                                accs[k] = accs[k] + r
                                accm[k] = jnp.maximum(accm[k], r)
                        return (tuple(accs), tuple(accm))

                    accs, accm = lax.fori_loop(
                        0, NB, _ublk, (accs0, accm0))
                    for k in range(NVEC):
                        plsc.addupdate(
                            accv.at[pl.ds(LANE * k, LANE)], accs[k])
                        accv[pl.ds(FEAT + LANE * k, LANE)] = accm[k]
                    plsc.addupdate(
                        accv.at[pl.ds(2 * FEAT, LANE)],
                        jnp.full((LANE,), float(RCHUNK), jnp.float32))

                def _blk(blk, _, buf=buf, boff=boff, sidx0=sidx0,
                         valid=valid):
                    rem = valid - blk * LANE
                    sldx = jnp.minimum(sidx0 + blk * LANE, c)
                    svec = segv[pl.ds(sldx, LANE)]
                    s = svec[0]
                    fast = jnp.logical_and(svec[LANE - 1] == s, rem >= LANE)

                    @pl.when(fast)
                    def _():
                        @pl.when(s != curs[0])
                        def _():
                            _flush_reset(s)

                        accs = [zero] * NVEC
                        accm = [accv[pl.ds(FEAT + LANE * k, LANE)]
                                for k in range(NVEC)]
                        base = (boff + blk * LANE) * FEAT
                        for i in range(LANE):
                            for k in range(NVEC):
                                r = buf[pl.ds(base + i * FEAT + k * LANE,
                                              LANE)]
                                accs[k] = accs[k] + r
                                accm[k] = jnp.maximum(accm[k], r)
                        for k in range(NVEC):
                            plsc.addupdate(
                                accv.at[pl.ds(LANE * k, LANE)], accs[k])
                            accv[pl.ds(FEAT + LANE * k, LANE)] = accm[k]
                        plsc.addupdate(
                            accv.at[pl.ds(2 * FEAT, LANE)],
                            jnp.full((LANE,), float(LANE), jnp.float32))

                    @pl.when(jnp.logical_not(fast))
                    def _():
                        def _row(r, _):
                            sr = segv[pl.ds(sidx0 + blk * LANE + r,
                                            LANE)][0]

                            @pl.when(sr != curs[0])
                            def _():
                                _flush_reset(sr)

                            base = (boff + blk * LANE + r) * FEAT
                            for k in range(NVEC):
                                rv = buf[pl.ds(base + LANE * k, LANE)]
                                plsc.addupdate(
                                    accv.at[pl.ds(LANE * k, LANE)], rv)
                                accv[pl.ds(FEAT + LANE * k, LANE)] = \
                                    jnp.maximum(
                                        accv[pl.ds(FEAT + LANE * k, LANE)],
                                        rv)
                            plsc.addupdate(
                                accv.at[pl.ds(2 * FEAT, LANE)], one)
                            return 0

                        lax.fori_loop(0, jnp.clip(rem, 0, LANE), _row, 0)

                    return 0

                @pl.when(jnp.logical_and(jnp.logical_not(uni), valid > 0))
                def _():
                    lax.fori_loop(0, NB, _blk, 0)

                cr = _ceff(j + NBUF)

                @pl.when(j + NBUF < nch)
                def _():
                    pltpu.async_copy(
                        x.at[pl.ds(cr * FEAT, RCHUNK * FEAT)],
                        bufs[b], sems[b])
            return 0

        lax.fori_loop(0, ngrp, _grp, 0)

        cur = curs[0]
        for k in range(NVEC):
            lsum[pl.ds(cur * FEAT + LANE * k, LANE)] = \
                accv[pl.ds(LANE * k, LANE)]
            lmax[pl.ds(cur * FEAT + LANE * k, LANE)] = \
                accv[pl.ds(FEAT + LANE * k, LANE)]
        lcnt[pl.ds(cur * LANE, LANE)] = accv[pl.ds(2 * FEAT, LANE)]

        pending.append((lsum, psum))
        pending.append((lmax, pmax))
        pending.append((lcnt, pcnt))
        pltpu.async_copy(lsum, psum.at[wid], psem)
        pltpu.async_copy(lmax, pmax.at[wid], psem)
        pltpu.async_copy(lcnt, pcnt.at[wid], psem)

    for (src, dst) in pending:
        pltpu.make_async_copy(src, dst.at[wid], psem).wait()


@functools.cache
def _make_sc_call():
    mesh = plsc.VectorSubcoreMesh(core_axis_name="c", subcore_axis_name="s")
    f32 = jnp.float32
    outs = []
    for _n in SIZES:
        outs += [jax.ShapeDtypeStruct((NW, NSEG * FEAT), f32),
                 jax.ShapeDtypeStruct((NW, NSEG * FEAT), f32),
                 jax.ShapeDtypeStruct((NW, NSEG * LANE), f32)]
    scratch = [
        pltpu.VMEM((RCHUNK * FEAT,), f32),
        pltpu.VMEM((RCHUNK * FEAT,), f32),
        pltpu.VMEM((_CFGS[0][0] + LANE,), jnp.int32),
        pltpu.VMEM((_CFGS[1][0] + LANE,), jnp.int32),
        pltpu.VMEM((_CFGS[2][0] + LANE,), jnp.int32),
        pltpu.VMEM((NSEG * FEAT,), f32),
        pltpu.VMEM((NSEG * FEAT,), f32),
        pltpu.VMEM((NSEG * LANE,), f32),
        pltpu.VMEM((NSEG * FEAT,), f32),
        pltpu.VMEM((NSEG * FEAT,), f32),
        pltpu.VMEM((NSEG * LANE,), f32),
        pltpu.VMEM((NSEG * FEAT,), f32),
        pltpu.VMEM((NSEG * FEAT,), f32),
        pltpu.VMEM((NSEG * LANE,), f32),
        pltpu.VMEM((2 * FEAT + LANE,), f32),
        pltpu.SMEM((1,), jnp.int32),
        pltpu.SemaphoreType.DMA,
        pltpu.SemaphoreType.DMA,
        pltpu.SemaphoreType.DMA,
    ]
    return pl.kernel(_sc_body, mesh=mesh, out_type=outs, scratch_types=scratch)


def _merge_body(ps0, pm0, pc0, ps1, pm1, pc1, ps2, pm2, pc2, out_ref):
    parts = ((ps0, pm0, pc0), (ps1, pm1, pc1), (ps2, pm2, pc2))
    for i, (ps, pm, pc) in enumerate(parts):
        s = jnp.sum(ps[...], axis=0)
        m = jnp.max(pm[...], axis=0)
        cvec = jnp.sum(pc[...], axis=0)
        cnt = cvec[:, 0:1]
        mean = s / jnp.maximum(cnt, 1.0)
        m = jnp.where(cnt > 0.0, m, 0.0)
        out_ref[:, i * FEAT:(i + 1) * FEAT] = m
        out_ref[:, 3 * FEAT + i * FEAT:3 * FEAT + (i + 1) * FEAT] = s
        out_ref[:, 6 * FEAT + i * FEAT:6 * FEAT + (i + 1) * FEAT] = mean


_merge = pl.pallas_call(
    _merge_body,
    out_shape=jax.ShapeDtypeStruct((NSEG, 9 * FEAT), jnp.float32),
)


def kernel(hs_0, hs_1, hs_2, gi_0, gi_1, gi_2):
    parts = _make_sc_call()(
        hs_0.reshape(-1), hs_1.reshape(-1), hs_2.reshape(-1),
        gi_2, gi_1, gi_0)
    shaped = []
    for i, p in enumerate(parts):
        if i % 3 == 2:
            shaped.append(p.reshape(NW, NSEG, LANE))
        else:
            shaped.append(p.reshape(NW, NSEG, FEAT))
    return _merge(*shaped)


# speedup vs baseline: 1.3500x; 1.1560x over previous
import functools

import jax
import jax.numpy as jnp
from jax import lax
from jax.experimental import pallas as pl
from jax.experimental.pallas import tpu as pltpu
from jax.experimental.pallas import tpu_sc as plsc

NSEG = 64
FEAT = 128
LANE = 16
NVEC = FEAT // LANE
NCORE = 2
NSUB = 16
NW = NCORE * NSUB
RCHUNK = 256
NBUF = 2
TCB = 5000

SIZES = (100000, 50000, 25000)


def _cfg(n):
    c = -(-n // NW)
    c = -(-c // 8) * 8
    nch = -(-c // RCHUNK)
    return c, nch


_CFGS = tuple(_cfg(n) for n in SIZES)


def _sc_body(x0, x1, x2, s0, s1, s2,
             pm0, pm1, pm2,
             buf0, buf1, sg0, sg1, sg2, lmax0, lmax1, lmax2,
             accv, curs, sem0, sem1, psem):
    wid = lax.axis_index("c") * NSUB + lax.axis_index("s")
    bufs = (buf0, buf1)
    sems = (sem0, sem1)
    ninf = jnp.full((LANE,), -jnp.inf, jnp.float32)

    layers = (
        (x0, s0, sg0, pm0, lmax0, SIZES[0], _CFGS[0]),
        (x1, s1, sg1, pm1, lmax1, SIZES[1], _CFGS[1]),
        (x2, s2, sg2, pm2, lmax2, SIZES[2], _CFGS[2]),
    )

    NB = RCHUNK // LANE

    for (_x, seg, segv, _pm, _lm, n, (c, nch)) in layers:
        start = wid * c
        sstart = jnp.minimum(start, n - c)
        pltpu.async_copy(seg.at[pl.ds(sstart, c)], segv.at[pl.ds(0, c)],
                         psem)
    for (_x, seg, segv, _pm, _lm, n, (c, nch)) in layers:
        start = wid * c
        sstart = jnp.minimum(start, n - c)
        pltpu.make_async_copy(seg.at[pl.ds(sstart, c)],
                              segv.at[pl.ds(0, c)], psem).wait()

    pending = []
    for (x, seg, segv, pmax, lmax, n, (c, nch)) in layers:
        def _init(i, _, lmax=lmax):
            lmax[pl.ds(i * LANE, LANE)] = ninf
            return 0
        lax.fori_loop(0, NSEG * NVEC, _init, 0)

        start = wid * c
        cnt = jnp.minimum(c, n - start)
        sstart = jnp.minimum(start, n - c)

        def _ceff(j, start=start, n=n):
            return jnp.minimum(start + j * RCHUNK, n - RCHUNK)

        def _flush_reset(new_seg, lmax=lmax):
            cur = curs[0]
            for k in range(NVEC):
                lmax[pl.ds(cur * FEAT + LANE * k, LANE)] = \
                    accv[pl.ds(LANE * k, LANE)]
                accv[pl.ds(LANE * k, LANE)] = ninf
            curs[0] = new_seg

        for k in range(NVEC):
            accv[pl.ds(LANE * k, LANE)] = ninf
        curs[0] = segv[pl.ds(start - sstart, LANE)][0]

        for b in range(min(NBUF, nch)):
            pltpu.async_copy(
                x.at[pl.ds(_ceff(b) * FEAT, RCHUNK * FEAT)],
                bufs[b], sems[b])

        ngrp = -(-nch // NBUF)

        def _grp(jj, _, x=x, segv=segv, c=c, n=n, start=start,
                 cnt=cnt, sstart=sstart, nch=nch, _ceff=_ceff,
                 _flush_reset=_flush_reset):
            for b in range(NBUF):
                j = NBUF * jj + b
                ce = _ceff(j)

                @pl.when(j < nch)
                def _():
                    pltpu.make_async_copy(
                        x.at[pl.ds(ce * FEAT, RCHUNK * FEAT)],
                        bufs[b], sems[b]).wait()

                buf = bufs[b]
                boff = start + j * RCHUNK - ce
                sidx0 = start + j * RCHUNK - sstart
                valid = jnp.clip(cnt - j * RCHUNK, 0, RCHUNK)

                sa = segv[pl.ds(jnp.minimum(sidx0, c), LANE)][0]
                sb = segv[pl.ds(jnp.clip(sidx0 + valid - 1, 0, c),
                                LANE)][0]
                uni = jnp.logical_and(valid == RCHUNK, sa == sb)

                @pl.when(uni)
                def _(buf=buf, boff=boff, sa=sa):
                    @pl.when(sa != curs[0])
                    def _():
                        _flush_reset(sa)

                    accm0 = tuple(accv[pl.ds(LANE * k, LANE)]
                                  for k in range(NVEC))

                    def _ublk(blk, accm, buf=buf, boff=boff):
                        accm = list(accm)
                        base = (boff + blk * LANE) * FEAT
                        for i in range(LANE):
                            for k in range(NVEC):
                                r = buf[pl.ds(base + i * FEAT + k * LANE,
                                              LANE)]
                                accm[k] = jnp.maximum(accm[k], r)
                        return tuple(accm)

                    accm = lax.fori_loop(0, NB, _ublk, accm0)
                    for k in range(NVEC):
                        accv[pl.ds(LANE * k, LANE)] = accm[k]

                def _blk(blk, _, buf=buf, boff=boff, sidx0=sidx0,
                         valid=valid):
                    rem = valid - blk * LANE
                    sldx = jnp.minimum(sidx0 + blk * LANE, c)
                    svec = segv[pl.ds(sldx, LANE)]
                    s = svec[0]
                    fast = jnp.logical_and(svec[LANE - 1] == s,
                                           rem >= LANE)

                    @pl.when(fast)
                    def _():
                        @pl.when(s != curs[0])
                        def _():
                            _flush_reset(s)

                        accm = [accv[pl.ds(LANE * k, LANE)]
                                for k in range(NVEC)]
                        base = (boff + blk * LANE) * FEAT
                        for i in range(LANE):
                            for k in range(NVEC):
                                r = buf[pl.ds(base + i * FEAT + k * LANE,
                                              LANE)]
                                accm[k] = jnp.maximum(accm[k], r)
                        for k in range(NVEC):
                            accv[pl.ds(LANE * k, LANE)] = accm[k]

                    @pl.when(jnp.logical_not(fast))
                    def _():
                        def _row(r, _):
                            sr = segv[pl.ds(sidx0 + blk * LANE + r,
                                            LANE)][0]

                            @pl.when(sr != curs[0])
                            def _():
                                _flush_reset(sr)

                            base = (boff + blk * LANE + r) * FEAT
                            for k in range(NVEC):
                                rv = buf[pl.ds(base + LANE * k, LANE)]
                                accv[pl.ds(LANE * k, LANE)] = \
                                    jnp.maximum(
                                        accv[pl.ds(LANE * k, LANE)], rv)
                            return 0

                        lax.fori_loop(0, jnp.clip(rem, 0, LANE), _row, 0)

                    return 0

                @pl.when(jnp.logical_and(jnp.logical_not(uni), valid > 0))
                def _():
                    lax.fori_loop(0, NB, _blk, 0)

                cr = _ceff(j + NBUF)

                @pl.when(j + NBUF < nch)
                def _():
                    pltpu.async_copy(
                        x.at[pl.ds(cr * FEAT, RCHUNK * FEAT)],
                        bufs[b], sems[b])
            return 0

        lax.fori_loop(0, ngrp, _grp, 0)

        cur = curs[0]
        for k in range(NVEC):
            lmax[pl.ds(cur * FEAT + LANE * k, LANE)] = \
                accv[pl.ds(LANE * k, LANE)]

        pending.append((lmax, pmax))
        pltpu.async_copy(lmax, pmax.at[wid], psem)

    for (src, dst) in pending:
        pltpu.make_async_copy(src, dst.at[wid], psem).wait()


@functools.cache
def _make_sc_call():
    mesh = plsc.VectorSubcoreMesh(core_axis_name="c", subcore_axis_name="s")
    f32 = jnp.float32
    outs = [jax.ShapeDtypeStruct((NW, NSEG * FEAT), f32)
            for _ in SIZES]
    scratch = [
        pltpu.VMEM((RCHUNK * FEAT,), f32),
        pltpu.VMEM((RCHUNK * FEAT,), f32),
        pltpu.VMEM((_CFGS[0][0] + LANE,), jnp.int32),
        pltpu.VMEM((_CFGS[1][0] + LANE,), jnp.int32),
        pltpu.VMEM((_CFGS[2][0] + LANE,), jnp.int32),
        pltpu.VMEM((NSEG * FEAT,), f32),
        pltpu.VMEM((NSEG * FEAT,), f32),
        pltpu.VMEM((NSEG * FEAT,), f32),
        pltpu.VMEM((FEAT,), f32),
        pltpu.SMEM((1,), jnp.int32),
        pltpu.SemaphoreType.DMA,
        pltpu.SemaphoreType.DMA,
        pltpu.SemaphoreType.DMA,
    ]
    return pl.kernel(_sc_body, mesh=mesh, out_type=outs,
                     scratch_types=scratch)


def _tc_sum_body(seg_ref, x_ref, ssum_ref, scnt_ref):
    pid = pl.program_id(0)

    @pl.when(pid == 0)
    def _():
        ssum_ref[...] = jnp.zeros_like(ssum_ref)
        scnt_ref[...] = jnp.zeros_like(scnt_ref)

    seg = seg_ref[0, 0, :]
    ids = lax.broadcasted_iota(jnp.int32, (NSEG, TCB), 0)
    mask = (seg[None, :] == ids).astype(jnp.float32)
    ssum_ref[...] += lax.dot_general(
        mask, x_ref[...], (((1,), (0,)), ((), ())),
        preferred_element_type=jnp.float32,
        precision=lax.Precision.HIGHEST)
    scnt_ref[...] += jnp.broadcast_to(
        jnp.sum(mask, axis=1, keepdims=True), (NSEG, FEAT))


@functools.cache
def _make_tc_sum(n):
    f32 = jnp.float32
    return pl.pallas_call(
        _tc_sum_body,
        grid=(n // TCB,),
        in_specs=[
            pl.BlockSpec((1, 1, TCB), lambda i: (i, 0, 0)),
            pl.BlockSpec((TCB, FEAT), lambda i: (i, 0)),
        ],
        out_specs=[
            pl.BlockSpec((NSEG, FEAT), lambda i: (0, 0)),
            pl.BlockSpec((NSEG, FEAT), lambda i: (0, 0)),
        ],
        out_shape=[jax.ShapeDtypeStruct((NSEG, FEAT), f32),
                   jax.ShapeDtypeStruct((NSEG, FEAT), f32)],
    )


def _merge_body(ss0, sc0, pm0, ss1, sc1, pm1, ss2, sc2, pm2, out_ref):
    parts = ((ss0, sc0, pm0), (ss1, sc1, pm1), (ss2, sc2, pm2))
    for i, (ss, sc, pm) in enumerate(parts):
        s = ss[...]
        m = jnp.max(pm[...], axis=0)
        cnt = sc[:, 0:1]
        mean = s / jnp.maximum(cnt, 1.0)
        m = jnp.where(cnt > 0.0, m, 0.0)
        out_ref[:, i * FEAT:(i + 1) * FEAT] = m
        out_ref[:, 3 * FEAT + i * FEAT:3 * FEAT + (i + 1) * FEAT] = s
        out_ref[:, 6 * FEAT + i * FEAT:6 * FEAT + (i + 1) * FEAT] = mean


_merge = pl.pallas_call(
    _merge_body,
    out_shape=jax.ShapeDtypeStruct((NSEG, 9 * FEAT), jnp.float32),
)


def kernel(hs_0, hs_1, hs_2, gi_0, gi_1, gi_2):
    pm = _make_sc_call()(
        hs_0.reshape(-1), hs_1.reshape(-1), hs_2.reshape(-1),
        gi_2, gi_1, gi_0)
    pm = [p.reshape(NW, NSEG, FEAT) for p in pm]
    args = []
    for x, g, n in ((hs_0, gi_2, SIZES[0]), (hs_1, gi_1, SIZES[1]),
                    (hs_2, gi_0, SIZES[2])):
        ss, sc = _make_tc_sum(n)(g.reshape(n // TCB, 1, TCB), x)
        args += [ss, sc]
    return _merge(args[0], args[1], pm[0],
                  args[2], args[3], pm[1],
                  args[4], args[5], pm[2])
